# baseline (speedup 1.0000x reference)
import functools

import jax
import jax.numpy as jnp
from jax import lax
from jax.experimental import pallas as pl
from jax.experimental.pallas import tpu as pltpu
from jax.experimental.pallas import tpu_sc as plsc

MAXW = 1.0
NC, NS, L = 2, 16, 16
NW = NC * NS
CHUNK = 80


def _pre(pred, data):
    n, d = pred.shape
    h = d // 2

    def rne16(x):
        b = lax.bitcast_convert_type(x, jnp.int32)
        rb = jnp.bitwise_and(lax.shift_right_logical(b, 16), 1)
        b = b + 32767 + rb
        return lax.shift_right_logical(b, 16)

    def body(pred_ref, data_ref, diff_ref, acc_ref):
        df = pred_ref[...] - data_ref[...]
        acc_ref[0, 0] = jnp.sum(df * df)
        lo = rne16(df[:, :h])
        hi = rne16(df[:, h:])
        diff_ref[...] = jnp.bitwise_or(lo, lax.shift_left(hi, 16))

    return pl.pallas_call(
        body,
        out_shape=[
            jax.ShapeDtypeStruct((n, h), jnp.int32),
            jax.ShapeDtypeStruct((1, 1), jnp.float32),
        ],
        out_specs=[
            pl.BlockSpec(memory_space=pltpu.VMEM),
            pl.BlockSpec(memory_space=pltpu.SMEM),
        ],
    )(pred, data)


def _sc_edges(diff, src, dst, attr, n, e):
    epw = e // NW
    nchunk = epw // CHUNK
    groups = CHUNK // L

    mesh = plsc.VectorSubcoreMesh(core_axis_name="c", subcore_axis_name="s")

    @functools.partial(
        pl.kernel,
        out_type=jax.ShapeDtypeStruct((NW * n,), jnp.float32),
        mesh=mesh,
        compiler_params=pltpu.CompilerParams(
            needs_layout_passes=False, use_tc_tiling_on_sc=False),
        scratch_types=[
            pltpu.VMEM((epw,), jnp.int32),
            pltpu.VMEM((epw,), jnp.int32),
            pltpu.VMEM((epw,), jnp.float32),
            pltpu.VMEM((CHUNK, 64), jnp.int32),
            pltpu.VMEM((CHUNK, 64), jnp.int32),
            pltpu.VMEM((CHUNK, 64), jnp.int32),
            pltpu.VMEM((CHUNK, 64), jnp.int32),
            pltpu.VMEM((n,), jnp.float32),
            pltpu.VMEM((L,), jnp.float32),
            pltpu.SemaphoreType.DMA,
            pltpu.SemaphoreType.DMA,
        ],
    )
    def k(diff_hbm, src_hbm, dst_hbm, attr_hbm, out_hbm,
          sidx_v, didx_v, attr_v, srows0_v, drows0_v, srows1_v, drows1_v,
          nw_v, w16_v, sem0, sem1):
        c = lax.axis_index("c")
        s = lax.axis_index("s")
        wid = s * NC + c
        base = wid * epw

        pltpu.sync_copy(src_hbm.at[pl.ds(base, epw)], sidx_v)
        pltpu.sync_copy(dst_hbm.at[pl.ds(base, epw)], didx_v)
        pltpu.sync_copy(attr_hbm.at[pl.ds(base, epw)], attr_v)

        zeros = jnp.zeros((L,), jnp.float32)

        def zbody(i, carry):
            nw_v[pl.ds(i * L, L)] = zeros
            return carry

        lax.fori_loop(0, n // L, zbody, 0)

        iota = lax.iota(jnp.int32, L)
        neg_inf = jnp.full((2 * L,), -jnp.inf, jnp.bfloat16)
        lane15 = iota == (L - 1)
        himask = jnp.full((L,), -65536, jnp.int32)

        def issue(ci, srows_v, drows_v, sem):
            pltpu.async_copy(
                diff_hbm.at[sidx_v.at[pl.ds(ci * CHUNK, CHUNK)]], srows_v, sem)
            pltpu.async_copy(
                diff_hbm.at[didx_v.at[pl.ds(ci * CHUNK, CHUNK)]], drows_v, sem)

        def drain(srows_v, drows_v, sem):
            pltpu.make_async_copy(
                diff_hbm.at[sidx_v.at[pl.ds(0, CHUNK)]], srows_v, sem).wait()
            pltpu.make_async_copy(
                diff_hbm.at[didx_v.at[pl.ds(0, CHUNK)]], drows_v, sem).wait()

        def compute(ci, srows_v, drows_v):
            def group_body(g, gcarry):
                iw = L
                for eb in range(L // iw):
                    rows = [g * L + iw * eb + j for j in range(iw)]
                    ms = [neg_inf] * iw
                    for dd in range(128 // (2 * L)):
                        svs = [plsc.bitcast(
                            srows_v[r, pl.ds(dd * L, L)], jnp.bfloat16)
                            for r in rows]
                        tvs = [plsc.bitcast(
                            drows_v[r, pl.ds(dd * L, L)], jnp.bfloat16)
                            for r in rows]
                        ms = [jnp.maximum(m, sv - tv)
                              for m, sv, tv in zip(ms, svs, tvs)]
                    mis = [plsc.bitcast(m, jnp.int32) for m in ms]
                    mlos = [plsc.bitcast(jnp.left_shift(mi, 16), jnp.float32)
                            for mi in mis]
                    mhis = [plsc.bitcast(jnp.bitwise_and(mi, himask),
                                         jnp.float32) for mi in mis]
                    mscs = [plsc.cummax(jnp.maximum(a, b))
                            for a, b in zip(mlos, mhis)]
                    for j in range(iw):
                        plsc.store_scatter(
                            w16_v,
                            [jnp.full((L,), iw * eb + j, jnp.int32)],
                            mscs[j], mask=lane15)
                off = ci * CHUNK + g * L
                w = w16_v[...] / attr_v[pl.ds(off, L)]
                plsc.addupdate_scatter(nw_v, [didx_v[pl.ds(off, L)]], w)
                return gcarry

            lax.fori_loop(0, groups, group_body, 0)

        assert nchunk % 2 == 1
        issue(0, srows0_v, drows0_v, sem0)

        def pair_body(j, carry):
            c0 = 2 * j
            issue(c0 + 1, srows1_v, drows1_v, sem1)
            drain(srows0_v, drows0_v, sem0)
            compute(c0, srows0_v, drows0_v)
            issue(c0 + 2, srows0_v, drows0_v, sem0)
            drain(srows1_v, drows1_v, sem1)
            compute(c0 + 1, srows1_v, drows1_v)
            return carry

        lax.fori_loop(0, (nchunk - 1) // 2, pair_body, 0)
        drain(srows0_v, drows0_v, sem0)
        compute(nchunk - 1, srows0_v, drows0_v)
        pltpu.sync_copy(nw_v, out_hbm.at[pl.ds(wid * n, n)])

    return k(diff, src, dst, attr)


def _post(nw_parts, sqsum, n, d):
    def body(nw_ref, sq_ref, out_ref):
        nw = jnp.sum(nw_ref[...], axis=0)
        nw = jnp.minimum(nw, MAXW)
        out_ref[0, 0] = sq_ref[0, 0] / jnp.float32(n * d) * jnp.sum(nw)

    return pl.pallas_call(
        body,
        out_shape=jax.ShapeDtypeStruct((1, 1), jnp.float32),
        in_specs=[
            pl.BlockSpec(memory_space=pltpu.VMEM),
            pl.BlockSpec(memory_space=pltpu.SMEM),
        ],
        out_specs=pl.BlockSpec(memory_space=pltpu.SMEM),
    )(nw_parts, sqsum)


def kernel(pred, data, edge_index, edge_attr):
    n, d = pred.shape
    e = edge_index.shape[1]

    src = edge_index[0].astype(jnp.int32)
    dst = edge_index[1].astype(jnp.int32)
    attr = edge_attr.reshape(e)

    diff32, sqsum = _pre(pred, data)
    nw_flat = _sc_edges(diff32, src, dst, attr, n, e)
    out = _post(nw_flat.reshape(NW, n), sqsum, n, d)
    return out[0, 0]


# speedup vs baseline: 1.0234x; 1.0234x over previous
import functools

import jax
import jax.numpy as jnp
from jax import lax
from jax.experimental import pallas as pl
from jax.experimental.pallas import tpu as pltpu
from jax.experimental.pallas import tpu_sc as plsc

MAXW = 1.0
NC, NS, L = 2, 16, 16
NW = NC * NS
CHUNK = 80


def _pre(pred, data):
    n, d = pred.shape
    h = d // 2

    def rne16(x):
        b = lax.bitcast_convert_type(x, jnp.int32)
        rb = jnp.bitwise_and(lax.shift_right_logical(b, 16), 1)
        b = b + 32767 + rb
        return lax.shift_right_logical(b, 16)

    def body(pred_ref, data_ref, diff_ref, acc_ref):
        df = pred_ref[...] - data_ref[...]
        acc_ref[0, 0] = jnp.sum(df * df)
        lo = rne16(df[:, :h])
        hi = rne16(df[:, h:])
        diff_ref[...] = jnp.bitwise_or(lo, lax.shift_left(hi, 16))

    return pl.pallas_call(
        body,
        out_shape=[
            jax.ShapeDtypeStruct((n, h), jnp.int32),
            jax.ShapeDtypeStruct((1, 1), jnp.float32),
        ],
        out_specs=[
            pl.BlockSpec(memory_space=pltpu.VMEM),
            pl.BlockSpec(memory_space=pltpu.SMEM),
        ],
    )(pred, data)


def _sc_edges(diff, src, dst, attr, n, e):
    epw = e // NW
    nchunk = epw // CHUNK
    groups = CHUNK // L

    mesh = plsc.VectorSubcoreMesh(core_axis_name="c", subcore_axis_name="s")

    @functools.partial(
        pl.kernel,
        out_type=jax.ShapeDtypeStruct((NW * n,), jnp.float32),
        mesh=mesh,
        compiler_params=pltpu.CompilerParams(
            needs_layout_passes=False, use_tc_tiling_on_sc=False),
        scratch_types=[
            pltpu.VMEM((epw,), jnp.int32),
            pltpu.VMEM((epw,), jnp.int32),
            pltpu.VMEM((epw,), jnp.float32),
            pltpu.VMEM((CHUNK, 64), jnp.int32),
            pltpu.VMEM((CHUNK, 64), jnp.int32),
            pltpu.VMEM((CHUNK, 64), jnp.int32),
            pltpu.VMEM((CHUNK, 64), jnp.int32),
            pltpu.VMEM((n,), jnp.float32),
            pltpu.VMEM((L,), jnp.float32),
            pltpu.SemaphoreType.DMA,
            pltpu.SemaphoreType.DMA,
        ],
    )
    def k(diff_hbm, src_hbm, dst_hbm, attr_hbm, out_hbm,
          sidx_v, didx_v, attr_v, srows0_v, drows0_v, srows1_v, drows1_v,
          nw_v, w16_v, sem0, sem1):
        c = lax.axis_index("c")
        s = lax.axis_index("s")
        wid = s * NC + c
        base = wid * epw

        pltpu.sync_copy(src_hbm.at[pl.ds(base, epw)], sidx_v)
        pltpu.sync_copy(dst_hbm.at[pl.ds(base, epw)], didx_v)
        pltpu.sync_copy(attr_hbm.at[pl.ds(base, epw)], attr_v)

        zeros = jnp.zeros((L,), jnp.float32)

        def zbody(i, carry):
            nw_v[pl.ds(i * L, L)] = zeros
            return carry

        lax.fori_loop(0, n // L, zbody, 0)

        iota = lax.iota(jnp.int32, L)
        neg_inf = jnp.full((2 * L,), -jnp.inf, jnp.bfloat16)
        lane15 = iota == (L - 1)
        himask = jnp.full((L,), -65536, jnp.int32)

        def issue(ci, srows_v, drows_v, sem):
            pltpu.async_copy(
                diff_hbm.at[sidx_v.at[pl.ds(ci * CHUNK, CHUNK)]], srows_v, sem)
            pltpu.async_copy(
                diff_hbm.at[didx_v.at[pl.ds(ci * CHUNK, CHUNK)]], drows_v, sem)

        def drain(srows_v, drows_v, sem):
            pltpu.make_async_copy(
                diff_hbm.at[sidx_v.at[pl.ds(0, CHUNK)]], srows_v, sem).wait()
            pltpu.make_async_copy(
                diff_hbm.at[didx_v.at[pl.ds(0, CHUNK)]], drows_v, sem).wait()

        def compute(ci, srows_v, drows_v):
            def group_body(g, gcarry):
                iw = 8
                for eb in range(L // iw):
                    rows = [g * L + iw * eb + j for j in range(iw)]
                    ms = [neg_inf] * iw
                    for dd in range(128 // (2 * L)):
                        svs = [plsc.bitcast(
                            srows_v[r, pl.ds(dd * L, L)], jnp.bfloat16)
                            for r in rows]
                        tvs = [plsc.bitcast(
                            drows_v[r, pl.ds(dd * L, L)], jnp.bfloat16)
                            for r in rows]
                        ms = [jnp.maximum(m, sv - tv)
                              for m, sv, tv in zip(ms, svs, tvs)]
                    mis = [plsc.bitcast(m, jnp.int32) for m in ms]
                    mlos = [plsc.bitcast(jnp.left_shift(mi, 16), jnp.float32)
                            for mi in mis]
                    mhis = [plsc.bitcast(jnp.bitwise_and(mi, himask),
                                         jnp.float32) for mi in mis]
                    mscs = [plsc.cummax(jnp.maximum(a, b))
                            for a, b in zip(mlos, mhis)]
                    for j in range(iw):
                        plsc.store_scatter(
                            w16_v,
                            [jnp.full((L,), iw * eb + j, jnp.int32)],
                            mscs[j], mask=lane15)
                off = ci * CHUNK + g * L
                w = w16_v[...] / attr_v[pl.ds(off, L)]
                plsc.addupdate_scatter(nw_v, [didx_v[pl.ds(off, L)]], w)
                return gcarry

            lax.fori_loop(0, groups, group_body, 0)

        assert nchunk % 2 == 1
        issue(0, srows0_v, drows0_v, sem0)

        def pair_body(j, carry):
            c0 = 2 * j
            issue(c0 + 1, srows1_v, drows1_v, sem1)
            drain(srows0_v, drows0_v, sem0)
            compute(c0, srows0_v, drows0_v)
            issue(c0 + 2, srows0_v, drows0_v, sem0)
            drain(srows1_v, drows1_v, sem1)
            compute(c0 + 1, srows1_v, drows1_v)
            return carry

        lax.fori_loop(0, (nchunk - 1) // 2, pair_body, 0)
        drain(srows0_v, drows0_v, sem0)
        compute(nchunk - 1, srows0_v, drows0_v)
        pltpu.sync_copy(nw_v, out_hbm.at[pl.ds(wid * n, n)])

    return k(diff, src, dst, attr)


def _post(nw_parts, sqsum, n, d):
    def body(nw_ref, sq_ref, out_ref):
        nw = jnp.sum(nw_ref[...], axis=0)
        nw = jnp.minimum(nw, MAXW)
        out_ref[0, 0] = sq_ref[0, 0] / jnp.float32(n * d) * jnp.sum(nw)

    return pl.pallas_call(
        body,
        out_shape=jax.ShapeDtypeStruct((1, 1), jnp.float32),
        in_specs=[
            pl.BlockSpec(memory_space=pltpu.VMEM),
            pl.BlockSpec(memory_space=pltpu.SMEM),
        ],
        out_specs=pl.BlockSpec(memory_space=pltpu.SMEM),
    )(nw_parts, sqsum)


def kernel(pred, data, edge_index, edge_attr):
    n, d = pred.shape
    e = edge_index.shape[1]

    src = edge_index[0].astype(jnp.int32)
    dst = edge_index[1].astype(jnp.int32)
    attr = edge_attr.reshape(e)

    diff32, sqsum = _pre(pred, data)
    nw_flat = _sc_edges(diff32, src, dst, attr, n, e)
    out = _post(nw_flat.reshape(NW, n), sqsum, n, d)
    return out[0, 0]
